# baseline (speedup 1.0000x reference)
import functools

import jax
import jax.numpy as jnp
from jax import lax
from jax.experimental import pallas as pl
from jax.experimental.pallas import tpu as pltpu
from jax.experimental.pallas import tpu_sc as plsc

NUM_EMBEDDINGS = 1000000
EMBEDDING_DIM = 64
BATCH = 16384
FIELDS = 26
FIELDS_PAD = 32
ROW_PAD = 128

NC = 2
NS = 16
NW = NC * NS

B_TOTAL = BATCH * FIELDS_PAD
B_PER_W = B_TOTAL // NW
CHUNK = 256
CHUNKS_PER_W = B_PER_W // CHUNK

NBUF = 4
LAG = 2

HALF = 507904
TBLK = 16384
NBLK_IN = -(-NUM_EMBEDDINGS // TBLK) - 1


def _transpose_body(lo_ref, hi_ref, o_ref):
    o_ref[...] = jnp.concatenate([lo_ref[...].T, hi_ref[...].T], axis=1)


def _gather_body(table, idx, out, idx_v, bufs_v, gsem, wsem):
    cid = lax.axis_index("c")
    sid = lax.axis_index("s")
    wid = sid * NC + cid
    row0 = wid * B_PER_W

    pltpu.sync_copy(idx.at[pl.ds(wid * CHUNKS_PER_W, CHUNKS_PER_W)], idx_v)

    def start_gather(c, b):
        pltpu.async_copy(table.at[idx_v.at[c]], bufs_v.at[b], gsem.at[b])

    def wait_gather(c, b):
        pltpu.make_async_copy(table.at[idx_v.at[c]], bufs_v.at[b], gsem.at[b]).wait()

    def dst(c):
        return out.at[pl.ds(row0 + c * CHUNK, CHUNK), pl.ds(0, EMBEDDING_DIM)]

    def start_write(c, b):
        pltpu.async_copy(bufs_v.at[b], dst(c), wsem.at[b])

    def wait_write(c, b):
        pltpu.make_async_copy(bufs_v.at[b], dst(c), wsem.at[b]).wait()

    for c in range(NBUF):
        start_gather(c, c)
    for c in range(LAG):
        wait_gather(c, c)
        start_write(c, c)

    @pl.loop(NBUF, CHUNKS_PER_W, step=NBUF)
    def _(c0):
        for b in range(NBUF):
            c = c0 + b
            wait_write(c - NBUF, b)
            start_gather(c, b)
            cw = c - NBUF + LAG
            wait_gather(cw, cw % NBUF)
            start_write(cw, cw % NBUF)

    for c in range(CHUNKS_PER_W - NBUF + LAG, CHUNKS_PER_W):
        wait_gather(c, c % NBUF)
        start_write(c, c % NBUF)
    for c in range(CHUNKS_PER_W - NBUF, CHUNKS_PER_W):
        wait_write(c, c % NBUF)


@jax.jit
def _embedding_gather(x, embeddings):
    pairs = pl.pallas_call(
        _transpose_body,
        grid=(HALF // TBLK,),
        in_specs=[
            pl.BlockSpec((EMBEDDING_DIM, TBLK), lambda i: (0, i)),
            pl.BlockSpec(
                (EMBEDDING_DIM, TBLK),
                lambda i: (0, jnp.minimum(i + HALF // TBLK, NBLK_IN)),
            ),
        ],
        out_specs=pl.BlockSpec((TBLK, 2 * EMBEDDING_DIM), lambda i: (i, 0)),
        out_shape=jax.ShapeDtypeStruct((HALF, 2 * EMBEDDING_DIM), jnp.float32),
    )(embeddings.T, embeddings.T)
    table64 = pairs.reshape(2 * HALF, EMBEDDING_DIM)

    slot = jnp.arange(FIELDS_PAD, dtype=x.dtype)[None, :]
    filler = jnp.arange(BATCH, dtype=x.dtype)[:, None] * (FIELDS_PAD // 2) + slot
    idx_pad = jnp.where(
        slot < FIELDS, jnp.pad(x, ((0, 0), (0, FIELDS_PAD - FIELDS))), filler
    )
    idx_pad = jnp.where(
        idx_pad < HALF, 2 * idx_pad, 2 * (idx_pad - HALF) + 1
    )
    idx2d = idx_pad.reshape(B_TOTAL // CHUNK, CHUNK)

    mesh = plsc.VectorSubcoreMesh(core_axis_name="c", subcore_axis_name="s")
    k = functools.partial(
        pl.kernel,
        mesh=mesh,
        out_type=jax.ShapeDtypeStruct((B_TOTAL, ROW_PAD), jnp.float32),
        scratch_types=[
            pltpu.VMEM((CHUNKS_PER_W, CHUNK), jnp.int32),
            pltpu.VMEM((NBUF, CHUNK, EMBEDDING_DIM), jnp.float32),
            pltpu.SemaphoreType.DMA((NBUF,)),
            pltpu.SemaphoreType.DMA((NBUF,)),
        ],
        compiler_params=pltpu.CompilerParams(use_tc_tiling_on_sc=False),
    )(_gather_body)
    outp = k(table64, idx2d)
    return outp.reshape(BATCH, FIELDS_PAD, ROW_PAD)[:, :FIELDS, :EMBEDDING_DIM]


def kernel(x, embeddings):
    return _embedding_gather(x, embeddings)


# speedup vs baseline: 1.0023x; 1.0023x over previous
import functools

import jax
import jax.numpy as jnp
from jax import lax
from jax.experimental import pallas as pl
from jax.experimental.pallas import tpu as pltpu
from jax.experimental.pallas import tpu_sc as plsc

NUM_EMBEDDINGS = 1000000
EMBEDDING_DIM = 64
BATCH = 16384
FIELDS = 26
FIELDS_PAD = 32
ROW_PAD = 128

NC = 2
NS = 16
NW = NC * NS

B_TOTAL = BATCH * FIELDS_PAD
B_PER_W = B_TOTAL // NW
CHUNK = 128
CHUNKS_PER_W = B_PER_W // CHUNK

NBUF = 8
LAG = 4

HALF = 507904
TBLK = 16384
NBLK_IN = -(-NUM_EMBEDDINGS // TBLK) - 1


def _transpose_body(lo_ref, hi_ref, o_ref):
    o_ref[...] = jnp.concatenate([lo_ref[...].T, hi_ref[...].T], axis=1)


def _gather_body(table, idx, out, idx_v, bufs_v, gsem, wsem):
    cid = lax.axis_index("c")
    sid = lax.axis_index("s")
    wid = sid * NC + cid
    row0 = wid * B_PER_W

    pltpu.sync_copy(idx.at[pl.ds(wid * CHUNKS_PER_W, CHUNKS_PER_W)], idx_v)

    def start_gather(c, b):
        pltpu.async_copy(table.at[idx_v.at[c]], bufs_v.at[b], gsem.at[b])

    def wait_gather(c, b):
        pltpu.make_async_copy(table.at[idx_v.at[c]], bufs_v.at[b], gsem.at[b]).wait()

    def dst(c):
        return out.at[pl.ds(row0 + c * CHUNK, CHUNK), pl.ds(0, EMBEDDING_DIM)]

    def start_write(c, b):
        pltpu.async_copy(bufs_v.at[b], dst(c), wsem.at[b])

    def wait_write(c, b):
        pltpu.make_async_copy(bufs_v.at[b], dst(c), wsem.at[b]).wait()

    for c in range(NBUF):
        start_gather(c, c)
    for c in range(LAG):
        wait_gather(c, c)
        start_write(c, c)

    @pl.loop(NBUF, CHUNKS_PER_W, step=NBUF)
    def _(c0):
        for b in range(NBUF):
            c = c0 + b
            wait_write(c - NBUF, b)
            start_gather(c, b)
            cw = c - NBUF + LAG
            wait_gather(cw, cw % NBUF)
            start_write(cw, cw % NBUF)

    for c in range(CHUNKS_PER_W - NBUF + LAG, CHUNKS_PER_W):
        wait_gather(c, c % NBUF)
        start_write(c, c % NBUF)
    for c in range(CHUNKS_PER_W - NBUF, CHUNKS_PER_W):
        wait_write(c, c % NBUF)


@jax.jit
def _embedding_gather(x, embeddings):
    pairs = pl.pallas_call(
        _transpose_body,
        grid=(HALF // TBLK,),
        in_specs=[
            pl.BlockSpec((EMBEDDING_DIM, TBLK), lambda i: (0, i)),
            pl.BlockSpec(
                (EMBEDDING_DIM, TBLK),
                lambda i: (0, jnp.minimum(i + HALF // TBLK, NBLK_IN)),
            ),
        ],
        out_specs=pl.BlockSpec((TBLK, 2 * EMBEDDING_DIM), lambda i: (i, 0)),
        out_shape=jax.ShapeDtypeStruct((HALF, 2 * EMBEDDING_DIM), jnp.float32),
    )(embeddings.T, embeddings.T)
    table64 = pairs.reshape(2 * HALF, EMBEDDING_DIM)

    slot = jnp.arange(FIELDS_PAD, dtype=x.dtype)[None, :]
    filler = jnp.arange(BATCH, dtype=x.dtype)[:, None] * (FIELDS_PAD // 2) + slot
    idx_pad = jnp.where(
        slot < FIELDS, jnp.pad(x, ((0, 0), (0, FIELDS_PAD - FIELDS))), filler
    )
    idx_pad = jnp.where(
        idx_pad < HALF, 2 * idx_pad, 2 * (idx_pad - HALF) + 1
    )
    idx2d = idx_pad.reshape(B_TOTAL // CHUNK, CHUNK)

    mesh = plsc.VectorSubcoreMesh(core_axis_name="c", subcore_axis_name="s")
    k = functools.partial(
        pl.kernel,
        mesh=mesh,
        out_type=jax.ShapeDtypeStruct((B_TOTAL, ROW_PAD), jnp.float32),
        scratch_types=[
            pltpu.VMEM((CHUNKS_PER_W, CHUNK), jnp.int32),
            pltpu.VMEM((NBUF, CHUNK, EMBEDDING_DIM), jnp.float32),
            pltpu.SemaphoreType.DMA((NBUF,)),
            pltpu.SemaphoreType.DMA((NBUF,)),
        ],
        compiler_params=pltpu.CompilerParams(use_tc_tiling_on_sc=False),
    )(_gather_body)
    outp = k(table64, idx2d)
    return outp.reshape(BATCH, FIELDS_PAD, ROW_PAD)[:, :FIELDS, :EMBEDDING_DIM]


def kernel(x, embeddings):
    return _embedding_gather(x, embeddings)
